# baseline (speedup 1.0000x reference)
import jax
import jax.numpy as jnp
from jax import lax
from jax.experimental import pallas as pl
from jax.experimental.pallas import tpu as pltpu
from jax.experimental.pallas import tpu_sc as plsc

_NC = 2
_NS = 16
_NW = _NC * _NS
_K = 4
_NBUF = 3


def _gather_body(table_hbm, idx_hbm, out_hbm, idx_v, rows0, rows1, rows2,
                 g0, g1, g2):
    wid = lax.axis_index("s") * _NC + lax.axis_index("c")
    nchunk = idx_hbm.shape[1]
    n_seq = out_hbm.shape[1]
    w_per_b = n_seq // (nchunk * _K)
    pltpu.sync_copy(idx_hbm.at[wid], idx_v)
    rows = (rows0, rows1, rows2)
    gsems = (g0, g1, g2)
    bb = wid // w_per_b
    seq0 = (wid % w_per_b) * (nchunk * _K)

    def out_slice(chunk):
        return out_hbm.at[bb, pl.ds(seq0 + chunk * _K, _K)]

    for b in range(2):
        pltpu.async_copy(table_hbm.at[idx_v.at[b]], rows[b], gsems[b])

    n_main = ((nchunk - 1) // _NBUF) * _NBUF

    @pl.loop(0, n_main, step=_NBUF)
    def _(p):
        for b in range(_NBUF):
            i = p + b
            pltpu.make_async_copy(table_hbm.at[idx_v.at[i]], rows[b], gsems[b]).wait()
            nxt = i + 2
            bn = (b + 2) % _NBUF

            @pl.when(nxt < nchunk)
            def _():
                pltpu.async_copy(table_hbm.at[idx_v.at[nxt]], rows[bn], gsems[bn])

            pltpu.sync_copy(rows[b], out_slice(i))

    for t in range(nchunk - n_main):
        j = n_main + t
        bt = j % _NBUF
        pltpu.make_async_copy(table_hbm.at[idx_v.at[j]], rows[bt], gsems[bt]).wait()
        pltpu.sync_copy(rows[bt], out_slice(j))


def kernel(idx, table):
    b, s = idx.shape
    vocab, d = table.shape
    n_tok = b * s
    nchunk = n_tok // (_NW * _K)
    idx3 = idx.reshape(_NW, nchunk, _K).astype(jnp.int32)
    mesh = plsc.VectorSubcoreMesh(core_axis_name="c", subcore_axis_name="s")
    run = pl.kernel(
        _gather_body,
        out_type=jax.ShapeDtypeStruct((b, s, d), jnp.float32),
        mesh=mesh,
        scratch_types=[
            pltpu.VMEM((nchunk, _K), jnp.int32),
        ]
        + [pltpu.VMEM((_K, d), jnp.float32)] * _NBUF
        + [pltpu.SemaphoreType.DMA] * _NBUF,
    )
    return run(table, idx3)
